# baseline (speedup 1.0000x reference)
import functools

import jax
import jax.numpy as jnp
from jax import lax
from jax.experimental import pallas as pl
from jax.experimental.pallas import tpu as pltpu
from jax.experimental.pallas import tpu_sc as plsc

N_NODES = 10000
D = 128
E = 320000
NC = 2
NS = 16
NW = NC * NS
EPT = E // NW
K = 80
NCH = EPT // K
NSTG = 25
NCH_S = NCH // NSTG
ROWS_CO = 640
NCO_MAIN = ROWS_CO // K
NCO_LAST = (N_NODES - 15 * ROWS_CO) // K


def _make_sc_agg(compute_deg):
    mesh = plsc.VectorSubcoreMesh(core_axis_name="c", subcore_axis_name="s")
    nbuf = 3
    out_type = [jax.ShapeDtypeStruct((NC, N_NODES, D), jnp.float32)]
    if compute_deg:
        out_type.append(jax.ShapeDtypeStruct((NW, N_NODES), jnp.float32))
    scratch = [
        pltpu.VMEM((2, NCH_S, K), jnp.int32),
        pltpu.VMEM((2, NCH_S, K), jnp.int32),
        pltpu.VMEM((nbuf, K, D), jnp.float32),
        pltpu.VMEM_SHARED((N_NODES, D), jnp.float32),
        pltpu.SemaphoreType.DMA((nbuf,)),
        pltpu.SemaphoreType.DMA((2,)),
        pltpu.SemaphoreType.DMA((nbuf,)),
    ]
    if compute_deg:
        scratch.append(pltpu.VMEM((N_NODES,), jnp.float32))
        scratch.append(pltpu.VMEM((NCH_S * K,), jnp.int32))

    @functools.partial(pl.kernel, mesh=mesh, out_type=out_type,
                       scratch_types=scratch,
                       compiler_params=pltpu.CompilerParams(needs_layout_passes=False))
    def body(x_hbm, ei_hbm, *rest):
        dflat = None
        if compute_deg:
            (eif_hbm, z_hbm, zd_hbm, aggp_hbm, degp_hbm, src_v, dst_v, rows_v,
             agg_sh, gsem, isem, ssem, deg_v, dflat) = rest
        else:
            z_hbm, aggp_hbm, src_v, dst_v, rows_v, agg_sh, gsem, isem, ssem = rest
        src_hbm = ei_hbm.at[0]
        dst_hbm = ei_hbm.at[1]
        c = lax.axis_index("c")
        s = lax.axis_index("s")
        wid = c * NS + s
        co_base = s * ROWS_CO
        nco = jnp.where(s == NS - 1, NCO_LAST, NCO_MAIN)

        pltpu.sync_copy(z_hbm, rows_v.at[1])

        def zi(i, carry):
            pltpu.async_copy(rows_v.at[1],
                             agg_sh.at[pl.ds(co_base + i * K, K)], ssem.at[0])
            return carry

        lax.fori_loop(0, nco, zi, 0)

        pltpu.sync_copy(src_hbm.at[wid, 0], src_v.at[0])
        pltpu.sync_copy(dst_hbm.at[wid, 0], dst_v.at[0])
        pltpu.async_copy(x_hbm.at[src_v.at[0, 0]], rows_v.at[0], gsem.at[0])
        if compute_deg:
            pltpu.sync_copy(zd_hbm, deg_v)

        def zdrain(i, carry):
            pltpu.make_async_copy(rows_v.at[1], agg_sh.at[pl.ds(co_base, K)],
                                  ssem.at[0]).wait()
            return carry

        lax.fori_loop(0, nco, zdrain, 0)

        plsc.subcore_barrier()

        ones = jnp.ones((16,), jnp.float32)

        def stage_body(st, carry):
            p = lax.rem(st, 2)
            j0 = st * NCH_S

            @pl.when(st > 0)
            def _():
                for q in range(1, nbuf):
                    bq = lax.rem(j0 - q + nbuf, nbuf)
                    pltpu.make_async_copy(rows_v.at[bq],
                                          agg_sh.at[dst_v.at[p, 0]],
                                          ssem.at[bq]).wait()
                pltpu.make_async_copy(src_hbm.at[wid, 0], src_v.at[p],
                                      isem.at[0]).wait()
                pltpu.make_async_copy(dst_hbm.at[wid, 0], dst_v.at[p],
                                      isem.at[1]).wait()

            @pl.when(st + 1 < NSTG)
            def _():
                pltpu.async_copy(src_hbm.at[wid, st + 1], src_v.at[1 - p],
                                 isem.at[0])
                pltpu.async_copy(dst_hbm.at[wid, st + 1], dst_v.at[1 - p],
                                 isem.at[1])

            if compute_deg:
                pltpu.sync_copy(eif_hbm.at[1, wid, st], dflat)

            bh = lax.rem(j0, nbuf)

            @pl.when(st > 0)
            def _():
                pltpu.async_copy(x_hbm.at[src_v.at[p, 0]], rows_v.at[bh],
                                 gsem.at[bh])

            def chunk_body(jj, carry2):
                b = lax.rem(j0 + jj, nbuf)
                b1 = lax.rem(j0 + jj + 1, nbuf)

                @pl.when(jj >= nbuf - 1)
                def _():
                    pltpu.make_async_copy(rows_v.at[b1],
                                          agg_sh.at[dst_v.at[p, jj]],
                                          ssem.at[b1]).wait()

                @pl.when(jj + 1 < NCH_S)
                def _():
                    pltpu.async_copy(x_hbm.at[src_v.at[p, jj + 1]],
                                     rows_v.at[b1], gsem.at[b1])

                if compute_deg:
                    base = jj * K
                    for l in range(K // 16):
                        idx = dflat[pl.ds(base + l * 16, 16)]
                        plsc.addupdate_scatter(deg_v, [idx], ones)

                pltpu.make_async_copy(x_hbm.at[src_v.at[p, jj]], rows_v.at[b],
                                      gsem.at[b]).wait()
                pltpu.async_copy(rows_v.at[b], agg_sh.at[dst_v.at[p, jj]],
                                 ssem.at[b], add=True)
                return carry2

            lax.fori_loop(0, NCH_S, chunk_body, 0)
            return carry

        lax.fori_loop(0, NSTG, stage_body, 0)

        for q in range(nbuf - 1):
            bq = (NCH - 1 - q) % nbuf
            pltpu.make_async_copy(rows_v.at[bq], agg_sh.at[dst_v.at[0, 0]],
                                  ssem.at[bq]).wait()

        plsc.subcore_barrier()

        if compute_deg:
            pltpu.sync_copy(deg_v, degp_hbm.at[wid])

        def co(i, carry):
            b = lax.rem(i, 2)

            @pl.when(i >= 2)
            def _():
                pltpu.make_async_copy(rows_v.at[b],
                                      aggp_hbm.at[c, pl.ds(co_base, K)],
                                      gsem.at[b]).wait()

            pltpu.sync_copy(agg_sh.at[pl.ds(co_base + i * K, K)], rows_v.at[b])
            pltpu.async_copy(rows_v.at[b], aggp_hbm.at[c, pl.ds(co_base + i * K, K)],
                             gsem.at[b])
            return carry

        lax.fori_loop(0, nco, co, 0)
        for b in range(2):
            pltpu.make_async_copy(rows_v.at[b], aggp_hbm.at[c, pl.ds(co_base, K)],
                                  gsem.at[b]).wait()

    return body


_sc_agg_deg = _make_sc_agg(True)
_sc_agg = _make_sc_agg(False)

BLK = 2000


def _tc_layer_body(relu, aggp_ref, degp_ref, x_ref, wl_ref, wr_ref, b_ref, o_ref):
    a = aggp_ref[0] + aggp_ref[1]
    deg = jnp.sum(degp_ref[...], axis=1)
    rdeg = 1.0 / jnp.maximum(deg, 1.0)
    mean = a * rdeg[:, None]
    acc = jnp.dot(mean, wl_ref[...], preferred_element_type=jnp.float32)
    acc = acc + jnp.dot(x_ref[...], wr_ref[...], preferred_element_type=jnp.float32)
    acc = acc + b_ref[...]
    if relu:
        acc = jnp.maximum(acc, 0.0)
    o_ref[...] = acc


def _tc_layer(aggp, degp, xin, Wl, b, Wr, relu):
    return pl.pallas_call(
        functools.partial(_tc_layer_body, relu),
        grid=(N_NODES // BLK,),
        in_specs=[
            pl.BlockSpec((NC, BLK, D), lambda i: (0, i, 0)),
            pl.BlockSpec((BLK, NW), lambda i: (i, 0)),
            pl.BlockSpec((BLK, D), lambda i: (i, 0)),
            pl.BlockSpec((D, D), lambda i: (0, 0)),
            pl.BlockSpec((D, D), lambda i: (0, 0)),
            pl.BlockSpec((1, D), lambda i: (0, 0)),
        ],
        out_specs=pl.BlockSpec((BLK, D), lambda i: (i, 0)),
        out_shape=jax.ShapeDtypeStruct((N_NODES, D), jnp.float32),
    )(aggp, degp, xin, Wl, Wr, b.reshape(1, D))


def kernel(x, edge_index, W0l, b0, W0r, W1l, b1, W1r):
    ei32 = edge_index.astype(jnp.int32)
    ei = ei32.reshape(2, NW, NSTG, NCH_S, K)
    eif = ei32.reshape(2, NW, NSTG, NCH_S * K)
    z = jnp.zeros((K, D), jnp.float32)
    zd = jnp.zeros((N_NODES,), jnp.float32)

    aggp1, degp = _sc_agg_deg(x, ei, eif, z, zd)
    degp = degp.T
    h = _tc_layer(aggp1, degp, x, W0l, b0, W0r, relu=True)
    aggp2 = _sc_agg(h, ei, z)
    if isinstance(aggp2, (list, tuple)):
        aggp2 = aggp2[0]
    out = _tc_layer(aggp2, degp, h, W1l, b1, W1r, relu=False)
    return out


# speedup vs baseline: 1.1406x; 1.1406x over previous
import functools

import jax
import jax.numpy as jnp
from jax import lax
from jax.experimental import pallas as pl
from jax.experimental.pallas import tpu as pltpu
from jax.experimental.pallas import tpu_sc as plsc

N_NODES = 10000
D = 128
E = 320000
NC = 2
NS = 16
NW = NC * NS
EPT = E // NW
K = 80
NCH = EPT // K
NSTG = 5
NCH_S = NCH // NSTG
ROWS_CO = 640
NCO_MAIN = ROWS_CO // K
NCO_LAST = (N_NODES - 15 * ROWS_CO) // K


def _make_sc_agg(compute_deg):
    mesh = plsc.VectorSubcoreMesh(core_axis_name="c", subcore_axis_name="s")
    nbuf = 2 if compute_deg else 3
    out_type = [jax.ShapeDtypeStruct((NC, N_NODES, D), jnp.float32)]
    if compute_deg:
        out_type.append(jax.ShapeDtypeStruct((NW, N_NODES), jnp.float32))
    scratch = [
        pltpu.VMEM((2, NCH_S, K), jnp.int32),
        pltpu.VMEM((2, NCH_S, K), jnp.int32),
        pltpu.VMEM((nbuf, K, D), jnp.float32),
        pltpu.VMEM_SHARED((N_NODES, D), jnp.float32),
        pltpu.SemaphoreType.DMA((nbuf,)),
        pltpu.SemaphoreType.DMA((2,)),
        pltpu.SemaphoreType.DMA((nbuf,)),
    ]
    if compute_deg:
        scratch.append(pltpu.VMEM((N_NODES,), jnp.float32))
        scratch.append(pltpu.VMEM((NCH_S * K,), jnp.int32))

    @functools.partial(pl.kernel, mesh=mesh, out_type=out_type,
                       scratch_types=scratch,
                       compiler_params=pltpu.CompilerParams(needs_layout_passes=False))
    def body(x_hbm, ei_hbm, *rest):
        dflat = None
        if compute_deg:
            (eif_hbm, z_hbm, zd_hbm, aggp_hbm, degp_hbm, src_v, dst_v, rows_v,
             agg_sh, gsem, isem, ssem, deg_v, dflat) = rest
        else:
            z_hbm, aggp_hbm, src_v, dst_v, rows_v, agg_sh, gsem, isem, ssem = rest
        src_hbm = ei_hbm.at[0]
        dst_hbm = ei_hbm.at[1]
        c = lax.axis_index("c")
        s = lax.axis_index("s")
        wid = c * NS + s
        co_base = s * ROWS_CO
        nco = jnp.where(s == NS - 1, NCO_LAST, NCO_MAIN)

        pltpu.sync_copy(z_hbm, rows_v.at[1])

        def zi(i, carry):
            pltpu.async_copy(rows_v.at[1],
                             agg_sh.at[pl.ds(co_base + i * K, K)], ssem.at[0])
            return carry

        lax.fori_loop(0, nco, zi, 0)

        pltpu.sync_copy(src_hbm.at[wid, 0], src_v.at[0])
        pltpu.sync_copy(dst_hbm.at[wid, 0], dst_v.at[0])
        pltpu.async_copy(x_hbm.at[src_v.at[0, 0]], rows_v.at[0], gsem.at[0])
        if compute_deg:
            pltpu.sync_copy(zd_hbm, deg_v)

        def zdrain(i, carry):
            pltpu.make_async_copy(rows_v.at[1], agg_sh.at[pl.ds(co_base, K)],
                                  ssem.at[0]).wait()
            return carry

        lax.fori_loop(0, nco, zdrain, 0)

        plsc.subcore_barrier()

        ones = jnp.ones((16,), jnp.float32)

        def stage_body(st, carry):
            p = lax.rem(st, 2)

            @pl.when(st > 0)
            def _():
                for q in range(1, nbuf):
                    bq = lax.rem(st - q + 2 * nbuf, nbuf)
                    pltpu.make_async_copy(rows_v.at[bq],
                                          agg_sh.at[dst_v.at[p, 0]],
                                          ssem.at[bq]).wait()
                pltpu.make_async_copy(src_hbm.at[wid, 0], src_v.at[p],
                                      isem.at[0]).wait()
                pltpu.make_async_copy(dst_hbm.at[wid, 0], dst_v.at[p],
                                      isem.at[1]).wait()

            @pl.when(st + 1 < NSTG)
            def _():
                pltpu.async_copy(src_hbm.at[wid, st + 1], src_v.at[1 - p],
                                 isem.at[0])
                pltpu.async_copy(dst_hbm.at[wid, st + 1], dst_v.at[1 - p],
                                 isem.at[1])

            if compute_deg:
                pltpu.sync_copy(eif_hbm.at[1, wid, st], dflat)

            bh = lax.rem(st, nbuf)

            @pl.when(st > 0)
            def _():
                pltpu.async_copy(x_hbm.at[src_v.at[p, 0]], rows_v.at[bh],
                                 gsem.at[bh])

            def chunk_body(jj, carry2):
                b = lax.rem(st + jj, nbuf)
                b1 = lax.rem(st + jj + 1, nbuf)

                @pl.when(jj >= nbuf - 1)
                def _():
                    pltpu.make_async_copy(rows_v.at[b1],
                                          agg_sh.at[dst_v.at[p, jj]],
                                          ssem.at[b1]).wait()

                @pl.when(jj + 1 < NCH_S)
                def _():
                    pltpu.async_copy(x_hbm.at[src_v.at[p, jj + 1]],
                                     rows_v.at[b1], gsem.at[b1])

                if compute_deg:
                    base = jj * K
                    for l in range(K // 16):
                        idx = dflat[pl.ds(base + l * 16, 16)]
                        plsc.addupdate_scatter(deg_v, [idx], ones)

                pltpu.make_async_copy(x_hbm.at[src_v.at[p, jj]], rows_v.at[b],
                                      gsem.at[b]).wait()
                pltpu.async_copy(rows_v.at[b], agg_sh.at[dst_v.at[p, jj]],
                                 ssem.at[b], add=True)
                return carry2

            lax.fori_loop(0, NCH_S, chunk_body, 0)
            return carry

        lax.fori_loop(0, NSTG, stage_body, 0)

        for q in range(nbuf - 1):
            bq = (NSTG - 1 + NCH_S - 1 - q) % nbuf
            pltpu.make_async_copy(rows_v.at[bq], agg_sh.at[dst_v.at[0, 0]],
                                  ssem.at[bq]).wait()

        plsc.subcore_barrier()

        if compute_deg:
            pltpu.sync_copy(deg_v, degp_hbm.at[wid])

        def co(i, carry):
            b = lax.rem(i, 2)

            @pl.when(i >= 2)
            def _():
                pltpu.make_async_copy(rows_v.at[b],
                                      aggp_hbm.at[c, pl.ds(co_base, K)],
                                      gsem.at[b]).wait()

            pltpu.sync_copy(agg_sh.at[pl.ds(co_base + i * K, K)], rows_v.at[b])
            pltpu.async_copy(rows_v.at[b], aggp_hbm.at[c, pl.ds(co_base + i * K, K)],
                             gsem.at[b])
            return carry

        lax.fori_loop(0, nco, co, 0)
        for b in range(2):
            pltpu.make_async_copy(rows_v.at[b], aggp_hbm.at[c, pl.ds(co_base, K)],
                                  gsem.at[b]).wait()

    return body


_sc_agg_deg = _make_sc_agg(True)
_sc_agg = _make_sc_agg(False)

BLK = 2000


def _tc_layer_body(relu, aggp_ref, degp_ref, x_ref, wl_ref, wr_ref, b_ref, o_ref):
    a = aggp_ref[0] + aggp_ref[1]
    deg = jnp.sum(degp_ref[...], axis=1)
    rdeg = 1.0 / jnp.maximum(deg, 1.0)
    mean = a * rdeg[:, None]
    acc = jnp.dot(mean, wl_ref[...], preferred_element_type=jnp.float32)
    acc = acc + jnp.dot(x_ref[...], wr_ref[...], preferred_element_type=jnp.float32)
    acc = acc + b_ref[...]
    if relu:
        acc = jnp.maximum(acc, 0.0)
    o_ref[...] = acc


def _tc_layer(aggp, degp, xin, Wl, b, Wr, relu):
    return pl.pallas_call(
        functools.partial(_tc_layer_body, relu),
        grid=(N_NODES // BLK,),
        in_specs=[
            pl.BlockSpec((NC, BLK, D), lambda i: (0, i, 0)),
            pl.BlockSpec((BLK, NW), lambda i: (i, 0)),
            pl.BlockSpec((BLK, D), lambda i: (i, 0)),
            pl.BlockSpec((D, D), lambda i: (0, 0)),
            pl.BlockSpec((D, D), lambda i: (0, 0)),
            pl.BlockSpec((1, D), lambda i: (0, 0)),
        ],
        out_specs=pl.BlockSpec((BLK, D), lambda i: (i, 0)),
        out_shape=jax.ShapeDtypeStruct((N_NODES, D), jnp.float32),
    )(aggp, degp, xin, Wl, Wr, b.reshape(1, D))


def kernel(x, edge_index, W0l, b0, W0r, W1l, b1, W1r):
    ei32 = edge_index.astype(jnp.int32)
    ei = ei32.reshape(2, NW, NSTG, NCH_S, K)
    eif = ei32.reshape(2, NW, NSTG, NCH_S * K)
    z = jnp.zeros((K, D), jnp.float32)
    zd = jnp.zeros((N_NODES,), jnp.float32)

    aggp1, degp = _sc_agg_deg(x, ei, eif, z, zd)
    degp = degp.T
    h = _tc_layer(aggp1, degp, x, W0l, b0, W0r, relu=True)
    aggp2 = _sc_agg(h, ei, z)
    if isinstance(aggp2, (list, tuple)):
        aggp2 = aggp2[0]
    out = _tc_layer(aggp2, degp, h, W1l, b1, W1r, relu=False)
    return out
